# baseline (speedup 1.0000x reference)
import functools
import jax
import jax.numpy as jnp
from jax.experimental import pallas as pl

B, S, D = 1, 2048, 1024
HD = 64
STD_Q, STD_KV = 12, 3
HSA_H = 4
CHUNK, TOPK = 64, 8
SW, HSW = 512, 512
SCALE = HD ** -0.5
NC = S // CHUNK
QB = 256
NQB = S // QB
NH = STD_Q + STD_KV * 2 + HSA_H + 2
NEG = -1e9


def _rot_half(x):
    return jnp.concatenate([-x[:, HD // 2:], x[:, :HD // 2]], axis=1)


def _proj_kernel(x_ref, wt_ref, cos_ref, sin_ref, qn_ref, kn_ref, out_ref):
    x = x_ref[...]
    y = jnp.dot(x, wt_ref[...], preferred_element_type=jnp.float32)
    cos = cos_ref[...]
    sin = sin_ref[...]
    qn = qn_ref[...]
    kn = kn_ref[...]
    for h in range(NH):
        seg = y[:, h * HD:(h + 1) * HD]
        is_q = h < STD_Q or (18 <= h < 22)
        is_k = (STD_Q <= h < STD_Q + STD_KV) or h == 22
        if is_q or is_k:
            w = qn if is_q else kn
            seg = seg * jax.lax.rsqrt(
                jnp.mean(seg * seg, axis=1, keepdims=True) + 1e-6) * w
            seg = seg * cos + _rot_half(seg) * sin
        out_ref[h, :, :] = seg


def _sel_kernel(hk_ref, hq0, hq1, hq2, hq3, out_ref, *, qb_id_unused=None):
    qb = pl.program_id(0)
    hk = hk_ref[0]
    r = jax.lax.broadcasted_iota(jnp.int32, (NC, S), 0)
    c = jax.lax.broadcasted_iota(jnp.int32, (NC, S), 1)
    A = jnp.where(c // CHUNK == r, 1.0 / CHUNK, 0.0)
    lm = jnp.dot(A, hk, preferred_element_type=jnp.float32)

    i_glob = qb * QB + jax.lax.broadcasted_iota(jnp.int32, (QB, 1), 0)
    off = (i_glob - (HSW - 1)) // CHUNK
    cid = jax.lax.broadcasted_iota(jnp.int32, (QB, NC), 1)
    avail = cid < off

    for h, hq_ref in enumerate((hq0, hq1, hq2, hq3)):
        hq = hq_ref[0]
        x = jnp.dot(hq, lm.T, preferred_element_type=jnp.float32) * SCALE
        xm = jnp.where(avail, x, NEG)
        thr = jnp.full((QB,), -jnp.inf, dtype=jnp.float32)
        for cc in range(NC):
            xc = xm[:, cc]
            cnt = jnp.sum((xm >= xc[:, None]).astype(jnp.int32), axis=1)
            thr = jnp.maximum(thr, jnp.where(cnt >= TOPK, xc, -jnp.inf))
        sel = (xm >= thr[:, None]) & avail
        out_ref[h, :, :] = sel.astype(jnp.float32)


def _std_attn_kernel(q_ref, k0, k1, k2, v0, v1, v2, out_ref):
    qb = pl.program_id(1)
    q = q_ref[0]
    i_glob = qb * QB + jax.lax.broadcasted_iota(jnp.int32, (QB, QB), 0)
    parts = []
    for t, k_ref in enumerate((k0, k1, k2)):
        kb = jnp.maximum(qb - 2 + t, 0)
        valid = (qb - 2 + t) >= 0
        kt = k_ref[0]
        lg = jnp.dot(q, kt.T, preferred_element_type=jnp.float32) * SCALE
        j_glob = kb * QB + jax.lax.broadcasted_iota(jnp.int32, (QB, QB), 1)
        mask = valid & (j_glob <= i_glob) & (i_glob - j_glob < SW)
        parts.append(jnp.where(mask, lg, NEG))
    l = jnp.concatenate(parts, axis=1)
    m = jnp.max(l, axis=1, keepdims=True)
    p = jnp.exp(l - m)
    s = jnp.sum(p, axis=1, keepdims=True)
    acc = jnp.zeros((QB, HD), dtype=jnp.float32)
    for t, v_ref in enumerate((v0, v1, v2)):
        acc = acc + jnp.dot(p[:, t * QB:(t + 1) * QB], v_ref[0],
                            preferred_element_type=jnp.float32)
    out_ref[0] = acc / s


def _hsa_attn_kernel(q_ref, k_ref, v_ref, sel_ref, out_ref):
    qb = pl.program_id(1)
    q = q_ref[0]
    k = k_ref[0]
    v = v_ref[0]
    sel = sel_ref[0]
    lg = jnp.dot(q, k.T, preferred_element_type=jnp.float32) * SCALE
    i_glob = qb * QB + jax.lax.broadcasted_iota(jnp.int32, (QB, S), 0)
    j_glob = jax.lax.broadcasted_iota(jnp.int32, (QB, S), 1)
    win = (j_glob <= i_glob) & (i_glob - j_glob < HSW)
    rr = jax.lax.broadcasted_iota(jnp.int32, (NC, S), 0)
    cc = jax.lax.broadcasted_iota(jnp.int32, (NC, S), 1)
    E = jnp.where(cc // CHUNK == rr, 1.0, 0.0)
    tok = jnp.dot(sel, E, preferred_element_type=jnp.float32) > 0.5
    mask = tok | win
    l = jnp.where(mask, lg, NEG)
    m = jnp.max(l, axis=1, keepdims=True)
    p = jnp.exp(l - m)
    s = jnp.sum(p, axis=1, keepdims=True)
    out_ref[0] = jnp.dot(p, v, preferred_element_type=jnp.float32) / s


def _outproj_kernel(x1_ref, x2_ref, w1_ref, w2_ref, out_ref):
    x1 = jnp.concatenate([x1_ref[h] for h in range(STD_Q)], axis=1)
    x2 = jnp.concatenate([x2_ref[h] for h in range(HSA_H)], axis=1)
    out_ref[...] = (
        jnp.dot(x1, w1_ref[...], preferred_element_type=jnp.float32)
        + jnp.dot(x2, w2_ref[...], preferred_element_type=jnp.float32))


def kernel(hidden_states, Wq, Wk, Wv, Whq, Whk, Whv, Wo, q_norm_w, k_norm_w):
    x = hidden_states.reshape(S, D)
    Wt = jnp.concatenate([Wq, Wk, Wv, Whq, Whk, Whv], axis=0).T

    pos = jnp.arange(S)
    inv = 1.0 / (10000.0 ** (jnp.arange(0, HD, 2).astype(jnp.float32) / HD))
    ang = pos[:, None] * inv[None, :]
    emb = jnp.concatenate([ang, ang], axis=-1)
    cos = jnp.cos(emb).astype(jnp.float32)
    sin = jnp.sin(emb).astype(jnp.float32)
    qn = q_norm_w.reshape(1, HD)
    kn = k_norm_w.reshape(1, HD)

    proj = pl.pallas_call(
        _proj_kernel,
        grid=(NQB,),
        in_specs=[
            pl.BlockSpec((QB, D), lambda i: (i, 0)),
            pl.BlockSpec((D, NH * HD), lambda i: (0, 0)),
            pl.BlockSpec((QB, HD), lambda i: (i, 0)),
            pl.BlockSpec((QB, HD), lambda i: (i, 0)),
            pl.BlockSpec((1, HD), lambda i: (0, 0)),
            pl.BlockSpec((1, HD), lambda i: (0, 0)),
        ],
        out_specs=pl.BlockSpec((NH, QB, HD), lambda i: (0, i, 0)),
        out_shape=jax.ShapeDtypeStruct((NH, S, HD), jnp.float32),
    )(x, Wt, cos, sin, qn, kn)

    sel = pl.pallas_call(
        _sel_kernel,
        grid=(NQB,),
        in_specs=[
            pl.BlockSpec((1, S, HD), lambda i: (22, 0, 0)),
            pl.BlockSpec((1, QB, HD), lambda i: (18, i, 0)),
            pl.BlockSpec((1, QB, HD), lambda i: (19, i, 0)),
            pl.BlockSpec((1, QB, HD), lambda i: (20, i, 0)),
            pl.BlockSpec((1, QB, HD), lambda i: (21, i, 0)),
        ],
        out_specs=pl.BlockSpec((HSA_H, QB, NC), lambda i: (0, i, 0)),
        out_shape=jax.ShapeDtypeStruct((HSA_H, S, NC), jnp.float32),
    )(proj, proj, proj, proj, proj)

    out_std = pl.pallas_call(
        _std_attn_kernel,
        grid=(STD_Q, NQB),
        in_specs=[
            pl.BlockSpec((1, QB, HD), lambda h, i: (h, i, 0)),
            pl.BlockSpec((1, QB, HD),
                         lambda h, i: (STD_Q + h // 4, jnp.maximum(i - 2, 0), 0)),
            pl.BlockSpec((1, QB, HD),
                         lambda h, i: (STD_Q + h // 4, jnp.maximum(i - 1, 0), 0)),
            pl.BlockSpec((1, QB, HD), lambda h, i: (STD_Q + h // 4, i, 0)),
            pl.BlockSpec((1, QB, HD),
                         lambda h, i: (15 + h // 4, jnp.maximum(i - 2, 0), 0)),
            pl.BlockSpec((1, QB, HD),
                         lambda h, i: (15 + h // 4, jnp.maximum(i - 1, 0), 0)),
            pl.BlockSpec((1, QB, HD), lambda h, i: (15 + h // 4, i, 0)),
        ],
        out_specs=pl.BlockSpec((1, QB, HD), lambda h, i: (h, i, 0)),
        out_shape=jax.ShapeDtypeStruct((STD_Q, S, HD), jnp.float32),
    )(proj, proj, proj, proj, proj, proj, proj)

    out_hsa = pl.pallas_call(
        _hsa_attn_kernel,
        grid=(HSA_H, NQB),
        in_specs=[
            pl.BlockSpec((1, QB, HD), lambda h, i: (18 + h, i, 0)),
            pl.BlockSpec((1, S, HD), lambda h, i: (22, 0, 0)),
            pl.BlockSpec((1, S, HD), lambda h, i: (23, 0, 0)),
            pl.BlockSpec((1, QB, NC), lambda h, i: (h, i, 0)),
        ],
        out_specs=pl.BlockSpec((1, QB, HD), lambda h, i: (h, i, 0)),
        out_shape=jax.ShapeDtypeStruct((HSA_H, S, HD), jnp.float32),
    )(proj, proj, proj, sel)

    WoT = Wo.T
    out = pl.pallas_call(
        _outproj_kernel,
        grid=(NQB,),
        in_specs=[
            pl.BlockSpec((STD_Q, QB, HD), lambda i: (0, i, 0)),
            pl.BlockSpec((HSA_H, QB, HD), lambda i: (0, i, 0)),
            pl.BlockSpec((STD_Q * HD, D), lambda i: (0, 0)),
            pl.BlockSpec((HSA_H * HD, D), lambda i: (0, 0)),
        ],
        out_specs=pl.BlockSpec((QB, D), lambda i: (i, 0)),
        out_shape=jax.ShapeDtypeStruct((S, D), jnp.float32),
    )(out_std, out_hsa, WoT[:STD_Q * HD], WoT[STD_Q * HD:])

    return out.reshape(B, S, D)


# speedup vs baseline: 1.3468x; 1.3468x over previous
import functools
import jax
import jax.numpy as jnp
from jax.experimental import pallas as pl

B, S, D = 1, 2048, 1024
HD = 64
STD_Q, STD_KV = 12, 3
HSA_H = 4
CHUNK, TOPK = 64, 8
SW, HSW = 512, 512
SCALE = HD ** -0.5
NC = S // CHUNK
QB = 256
NQB = S // QB
NH = STD_Q + STD_KV * 2 + HSA_H + 2
NEG = -1e9


def _rot_half(x):
    return jnp.concatenate([-x[:, HD // 2:], x[:, :HD // 2]], axis=1)


def _proj_kernel(x_ref, wt_ref, cos_ref, sin_ref, qn_ref, kn_ref, out_ref):
    x = x_ref[...]
    y = jnp.dot(x, wt_ref[...], preferred_element_type=jnp.float32)
    cos = cos_ref[...]
    sin = sin_ref[...]
    qn = qn_ref[...]
    kn = kn_ref[...]
    for h in range(NH):
        seg = y[:, h * HD:(h + 1) * HD]
        is_q = h < STD_Q or (18 <= h < 22)
        is_k = (STD_Q <= h < STD_Q + STD_KV) or h == 22
        if is_q or is_k:
            w = qn if is_q else kn
            seg = seg * jax.lax.rsqrt(
                jnp.mean(seg * seg, axis=1, keepdims=True) + 1e-6) * w
            seg = seg * cos + _rot_half(seg) * sin
        out_ref[h, :, :] = seg


def _sel_kernel(hk_ref, hq0, hq1, hq2, hq3, out_ref, *, qb_id_unused=None):
    qb = pl.program_id(0)
    hk = hk_ref[0]
    r = jax.lax.broadcasted_iota(jnp.int32, (NC, S), 0)
    c = jax.lax.broadcasted_iota(jnp.int32, (NC, S), 1)
    A = jnp.where(c // CHUNK == r, 1.0 / CHUNK, 0.0)
    lm = jnp.dot(A, hk, preferred_element_type=jnp.float32,
                 precision=jax.lax.Precision.HIGHEST)

    i_glob = qb * QB + jax.lax.broadcasted_iota(jnp.int32, (QB, 1), 0)
    off = (i_glob - (HSW - 1)) // CHUNK
    cid = jax.lax.broadcasted_iota(jnp.int32, (QB, NC), 1)
    avail = cid < off

    rowj = jax.lax.broadcasted_iota(jnp.int32, (NC, NC * NC), 0)
    colx = jax.lax.broadcasted_iota(jnp.int32, (NC, NC * NC), 1)
    Tm = jnp.where(colx % NC == rowj, 1.0, 0.0)
    Rm = jnp.where(colx // NC == rowj, 1.0, 0.0)

    for h, hq_ref in enumerate((hq0, hq1, hq2, hq3)):
        hq = hq_ref[0]
        x = jnp.dot(hq, lm.T, preferred_element_type=jnp.float32,
                    precision=jax.lax.Precision.HIGHEST) * SCALE
        xm = jnp.where(avail, x, NEG)
        a1 = jnp.dot(xm, Tm, preferred_element_type=jnp.float32,
                     precision=jax.lax.Precision.HIGHEST)
        a2 = jnp.dot(xm, Rm, preferred_element_type=jnp.float32,
                     precision=jax.lax.Precision.HIGHEST)
        ge = (a1 >= a2).astype(jnp.float32)
        cnt = jnp.dot(ge, Rm.T, preferred_element_type=jnp.float32,
                      precision=jax.lax.Precision.HIGHEST)
        thr = jnp.max(jnp.where(cnt >= TOPK, xm, -jnp.inf), axis=1, keepdims=True)
        sel = (xm >= thr) & avail
        out_ref[h, :, :] = sel.astype(jnp.float32)


def _std_attn_kernel(q_ref, k0, k1, k2, v0, v1, v2, out_ref):
    qb = pl.program_id(1)
    q = q_ref[0]
    i_glob = qb * QB + jax.lax.broadcasted_iota(jnp.int32, (QB, QB), 0)
    parts = []
    for t, k_ref in enumerate((k0, k1, k2)):
        kb = jnp.maximum(qb - 2 + t, 0)
        valid = (qb - 2 + t) >= 0
        kt = k_ref[0]
        lg = jnp.dot(q, kt.T, preferred_element_type=jnp.float32) * SCALE
        j_glob = kb * QB + jax.lax.broadcasted_iota(jnp.int32, (QB, QB), 1)
        mask = valid & (j_glob <= i_glob) & (i_glob - j_glob < SW)
        parts.append(jnp.where(mask, lg, NEG))
    l = jnp.concatenate(parts, axis=1)
    m = jnp.max(l, axis=1, keepdims=True)
    p = jnp.exp(l - m)
    s = jnp.sum(p, axis=1, keepdims=True)
    acc = jnp.zeros((QB, HD), dtype=jnp.float32)
    for t, v_ref in enumerate((v0, v1, v2)):
        acc = acc + jnp.dot(p[:, t * QB:(t + 1) * QB], v_ref[0],
                            preferred_element_type=jnp.float32)
    out_ref[0] = acc / s


def _hsa_attn_kernel(q_ref, k_ref, v_ref, sel_ref, out_ref):
    qb = pl.program_id(1)
    q = q_ref[0]
    k = k_ref[0]
    v = v_ref[0]
    sel = sel_ref[0]
    lg = jnp.dot(q, k.T, preferred_element_type=jnp.float32) * SCALE
    i_glob = qb * QB + jax.lax.broadcasted_iota(jnp.int32, (QB, S), 0)
    j_glob = jax.lax.broadcasted_iota(jnp.int32, (QB, S), 1)
    win = (j_glob <= i_glob) & (i_glob - j_glob < HSW)
    rr = jax.lax.broadcasted_iota(jnp.int32, (NC, S), 0)
    cc = jax.lax.broadcasted_iota(jnp.int32, (NC, S), 1)
    E = jnp.where(cc // CHUNK == rr, 1.0, 0.0)
    tok = jnp.dot(sel, E, preferred_element_type=jnp.float32) > 0.5
    mask = tok | win
    l = jnp.where(mask, lg, NEG)
    m = jnp.max(l, axis=1, keepdims=True)
    p = jnp.exp(l - m)
    s = jnp.sum(p, axis=1, keepdims=True)
    out_ref[0] = jnp.dot(p, v, preferred_element_type=jnp.float32) / s


def _outproj_kernel(x1_ref, x2_ref, w1_ref, w2_ref, out_ref):
    x1 = jnp.concatenate([x1_ref[h] for h in range(STD_Q)], axis=1)
    x2 = jnp.concatenate([x2_ref[h] for h in range(HSA_H)], axis=1)
    out_ref[...] = (
        jnp.dot(x1, w1_ref[...], preferred_element_type=jnp.float32)
        + jnp.dot(x2, w2_ref[...], preferred_element_type=jnp.float32))


def kernel(hidden_states, Wq, Wk, Wv, Whq, Whk, Whv, Wo, q_norm_w, k_norm_w):
    x = hidden_states.reshape(S, D)
    Wt = jnp.concatenate([Wq, Wk, Wv, Whq, Whk, Whv], axis=0).T

    pos = jnp.arange(S)
    inv = 1.0 / (10000.0 ** (jnp.arange(0, HD, 2).astype(jnp.float32) / HD))
    ang = pos[:, None] * inv[None, :]
    emb = jnp.concatenate([ang, ang], axis=-1)
    cos = jnp.cos(emb).astype(jnp.float32)
    sin = jnp.sin(emb).astype(jnp.float32)
    qn = q_norm_w.reshape(1, HD)
    kn = k_norm_w.reshape(1, HD)

    proj = pl.pallas_call(
        _proj_kernel,
        grid=(NQB,),
        in_specs=[
            pl.BlockSpec((QB, D), lambda i: (i, 0)),
            pl.BlockSpec((D, NH * HD), lambda i: (0, 0)),
            pl.BlockSpec((QB, HD), lambda i: (i, 0)),
            pl.BlockSpec((QB, HD), lambda i: (i, 0)),
            pl.BlockSpec((1, HD), lambda i: (0, 0)),
            pl.BlockSpec((1, HD), lambda i: (0, 0)),
        ],
        out_specs=pl.BlockSpec((NH, QB, HD), lambda i: (0, i, 0)),
        out_shape=jax.ShapeDtypeStruct((NH, S, HD), jnp.float32),
    )(x, Wt, cos, sin, qn, kn)

    sel = pl.pallas_call(
        _sel_kernel,
        grid=(NQB,),
        in_specs=[
            pl.BlockSpec((1, S, HD), lambda i: (22, 0, 0)),
            pl.BlockSpec((1, QB, HD), lambda i: (18, i, 0)),
            pl.BlockSpec((1, QB, HD), lambda i: (19, i, 0)),
            pl.BlockSpec((1, QB, HD), lambda i: (20, i, 0)),
            pl.BlockSpec((1, QB, HD), lambda i: (21, i, 0)),
        ],
        out_specs=pl.BlockSpec((HSA_H, QB, NC), lambda i: (0, i, 0)),
        out_shape=jax.ShapeDtypeStruct((HSA_H, S, NC), jnp.float32),
    )(proj, proj, proj, proj, proj)

    out_std = pl.pallas_call(
        _std_attn_kernel,
        grid=(STD_Q, NQB),
        in_specs=[
            pl.BlockSpec((1, QB, HD), lambda h, i: (h, i, 0)),
            pl.BlockSpec((1, QB, HD),
                         lambda h, i: (STD_Q + h // 4, jnp.maximum(i - 2, 0), 0)),
            pl.BlockSpec((1, QB, HD),
                         lambda h, i: (STD_Q + h // 4, jnp.maximum(i - 1, 0), 0)),
            pl.BlockSpec((1, QB, HD), lambda h, i: (STD_Q + h // 4, i, 0)),
            pl.BlockSpec((1, QB, HD),
                         lambda h, i: (15 + h // 4, jnp.maximum(i - 2, 0), 0)),
            pl.BlockSpec((1, QB, HD),
                         lambda h, i: (15 + h // 4, jnp.maximum(i - 1, 0), 0)),
            pl.BlockSpec((1, QB, HD), lambda h, i: (15 + h // 4, i, 0)),
        ],
        out_specs=pl.BlockSpec((1, QB, HD), lambda h, i: (h, i, 0)),
        out_shape=jax.ShapeDtypeStruct((STD_Q, S, HD), jnp.float32),
    )(proj, proj, proj, proj, proj, proj, proj)

    out_hsa = pl.pallas_call(
        _hsa_attn_kernel,
        grid=(HSA_H, NQB),
        in_specs=[
            pl.BlockSpec((1, QB, HD), lambda h, i: (18 + h, i, 0)),
            pl.BlockSpec((1, S, HD), lambda h, i: (22, 0, 0)),
            pl.BlockSpec((1, S, HD), lambda h, i: (23, 0, 0)),
            pl.BlockSpec((1, QB, NC), lambda h, i: (h, i, 0)),
        ],
        out_specs=pl.BlockSpec((1, QB, HD), lambda h, i: (h, i, 0)),
        out_shape=jax.ShapeDtypeStruct((HSA_H, S, HD), jnp.float32),
    )(proj, proj, proj, sel)

    WoT = Wo.T
    out = pl.pallas_call(
        _outproj_kernel,
        grid=(NQB,),
        in_specs=[
            pl.BlockSpec((STD_Q, QB, HD), lambda i: (0, i, 0)),
            pl.BlockSpec((HSA_H, QB, HD), lambda i: (0, i, 0)),
            pl.BlockSpec((STD_Q * HD, D), lambda i: (0, 0)),
            pl.BlockSpec((HSA_H * HD, D), lambda i: (0, 0)),
        ],
        out_specs=pl.BlockSpec((QB, D), lambda i: (i, 0)),
        out_shape=jax.ShapeDtypeStruct((S, D), jnp.float32),
    )(out_std, out_hsa, WoT[:STD_Q * HD], WoT[STD_Q * HD:])

    return out.reshape(B, S, D)


# speedup vs baseline: 1.8228x; 1.3534x over previous
import jax
import jax.numpy as jnp
from jax.experimental import pallas as pl

B, S, D = 1, 2048, 1024
HD = 64
STD_Q, STD_KV = 12, 3
HSA_H = 4
CHUNK, TOPK = 64, 8
SW, HSW = 512, 512
SCALE = HD ** -0.5
NC = S // CHUNK
QB = 256
NQB = S // QB
NH = STD_Q + STD_KV * 2 + HSA_H + 2
NEG = -1e9
HI = jax.lax.Precision.HIGHEST


def _rot_half(x):
    return jnp.concatenate([-x[:, HD // 2:], x[:, :HD // 2]], axis=1)


def _proj_kernel(x_ref, wt_ref, cos_ref, sin_ref, qn_ref, kn_ref, out_ref):
    x = x_ref[...]
    y = jnp.dot(x, wt_ref[...], preferred_element_type=jnp.float32)
    cos = cos_ref[...]
    sin = sin_ref[...]
    qn = qn_ref[...]
    kn = kn_ref[...]
    for h in range(NH):
        seg = y[:, h * HD:(h + 1) * HD]
        is_q = h < STD_Q or (18 <= h < 22)
        is_k = (STD_Q <= h < STD_Q + STD_KV) or h == 22
        if is_q or is_k:
            w = qn if is_q else kn
            seg = seg * jax.lax.rsqrt(
                jnp.mean(seg * seg, axis=1, keepdims=True) + 1e-6) * w
            seg = seg * cos + _rot_half(seg) * sin
        out_ref[h, :, :] = seg


def _fused_kernel(proj_ref, e_ref, tm_ref, rm_ref, rmt_ref, w3_ref, win_ref,
                  wot_ref, out_ref):
    qb = pl.program_id(0)
    rows = pl.ds(qb * QB, QB)

    hk = proj_ref[22]
    lm = jnp.dot(e_ref[...], hk, preferred_element_type=jnp.float32,
                 precision=HI) * (1.0 / CHUNK)
    hq_cat = jnp.concatenate(
        [proj_ref[18 + h, rows, :] for h in range(HSA_H)], axis=0)
    x = jnp.dot(hq_cat, lm.T, preferred_element_type=jnp.float32,
                precision=HI) * SCALE
    i_loc = qb * QB + jax.lax.broadcasted_iota(jnp.int32, (QB, 1), 0)
    off = (i_loc - (HSW - 1)) // CHUNK
    cid = jax.lax.broadcasted_iota(jnp.int32, (QB, NC), 1)
    avail = cid < off
    avail4 = jnp.concatenate([avail] * HSA_H, axis=0)
    xm = jnp.where(avail4, x, NEG)
    a1 = jnp.dot(xm, tm_ref[...], preferred_element_type=jnp.float32,
                 precision=HI)
    a2 = jnp.dot(xm, rm_ref[...], preferred_element_type=jnp.float32,
                 precision=HI)
    ge = (a1 >= a2).astype(jnp.float32)
    cnt = jnp.dot(ge, rmt_ref[...], preferred_element_type=jnp.float32,
                  precision=HI)
    thr = jnp.max(jnp.where(cnt >= TOPK, xm, -jnp.inf), axis=1, keepdims=True)
    sel = ((xm >= thr) & avail4).astype(jnp.float32)

    outs = []
    w3 = w3_ref[...]
    v0 = (qb >= 2).astype(jnp.float32)
    v1 = (qb >= 1).astype(jnp.float32)
    cm = jnp.concatenate([jnp.full((1, QB), v0, jnp.float32),
                          jnp.full((1, QB), v1, jnp.float32),
                          jnp.ones((1, QB), jnp.float32)], axis=1)
    kbs = [pl.ds(jnp.maximum(qb - 2 + t, 0) * QB, QB) for t in range(3)]
    for g in range(STD_KV):
        qcat = jnp.concatenate(
            [proj_ref[4 * g + hh, rows, :] for hh in range(4)], axis=0)
        kband = jnp.concatenate(
            [proj_ref[STD_Q + g, kb, :] for kb in kbs], axis=0)
        vband = jnp.concatenate(
            [proj_ref[STD_Q + STD_KV + g, kb, :] for kb in kbs], axis=0)
        lg = jnp.dot(qcat, kband.T,
                     preferred_element_type=jnp.float32) * SCALE
        m = jnp.max(lg, axis=1, keepdims=True)
        p = jnp.exp(lg - m) * w3 * cm
        s = jnp.sum(p, axis=1, keepdims=True)
        o = jnp.dot(p, vband, preferred_element_type=jnp.float32) / s
        outs.extend(o[hh * QB:(hh + 1) * QB] for hh in range(4))

    kf = proj_ref[22]
    vf = proj_ref[23]
    winblk = win_ref[...]
    for h in range(HSA_H):
        qh = hq_cat[h * QB:(h + 1) * QB]
        lg = jnp.dot(qh, kf.T, preferred_element_type=jnp.float32) * SCALE
        tok = jnp.dot(sel[h * QB:(h + 1) * QB], e_ref[...],
                      preferred_element_type=jnp.float32)
        msk = jnp.maximum(winblk, tok)
        m = jnp.max(lg, axis=1, keepdims=True)
        p = jnp.exp(lg - m) * msk
        s = jnp.sum(p, axis=1, keepdims=True)
        outs.append(jnp.dot(p, vf, preferred_element_type=jnp.float32) / s)

    xcat = jnp.concatenate(outs, axis=1)
    out_ref[...] = jnp.dot(xcat, wot_ref[...],
                           preferred_element_type=jnp.float32)


def kernel(hidden_states, Wq, Wk, Wv, Whq, Whk, Whv, Wo, q_norm_w, k_norm_w):
    x = hidden_states.reshape(S, D)
    Wt = jnp.concatenate([Wq, Wk, Wv, Whq, Whk, Whv], axis=0).T

    pos = jnp.arange(S)
    inv = 1.0 / (10000.0 ** (jnp.arange(0, HD, 2).astype(jnp.float32) / HD))
    ang = pos[:, None] * inv[None, :]
    emb = jnp.concatenate([ang, ang], axis=-1)
    cos = jnp.cos(emb).astype(jnp.float32)
    sin = jnp.sin(emb).astype(jnp.float32)
    qn = q_norm_w.reshape(1, HD)
    kn = k_norm_w.reshape(1, HD)

    proj = pl.pallas_call(
        _proj_kernel,
        grid=(NQB,),
        in_specs=[
            pl.BlockSpec((QB, D), lambda i: (i, 0)),
            pl.BlockSpec((D, NH * HD), lambda i: (0, 0)),
            pl.BlockSpec((QB, HD), lambda i: (i, 0)),
            pl.BlockSpec((QB, HD), lambda i: (i, 0)),
            pl.BlockSpec((1, HD), lambda i: (0, 0)),
            pl.BlockSpec((1, HD), lambda i: (0, 0)),
        ],
        out_specs=pl.BlockSpec((NH, QB, HD), lambda i: (0, i, 0)),
        out_shape=jax.ShapeDtypeStruct((NH, S, HD), jnp.float32),
    )(x, Wt, cos, sin, qn, kn)

    jj = jnp.arange(S)
    cidx = jnp.arange(NC)
    E = (jj[None, :] // CHUNK == cidx[:, None]).astype(jnp.float32)
    pair = jnp.arange(NC * NC)
    Tm = (pair[None, :] % NC == cidx[:, None]).astype(jnp.float32)
    Rm = (pair[None, :] // NC == cidx[:, None]).astype(jnp.float32)
    RmT = Rm.T
    r = jnp.arange(QB)
    col = jnp.arange(3 * QB)
    w3 = ((col[None, :] > r[:, None]) &
          (col[None, :] <= r[:, None] + SW)).astype(jnp.float32)
    w3x4 = jnp.tile(w3, (HSA_H, 1))
    ii = jnp.arange(S)
    win = ((jj[None, :] <= ii[:, None]) &
           (ii[:, None] - jj[None, :] < SW)).astype(jnp.float32)
    WoT = Wo.T

    out = pl.pallas_call(
        _fused_kernel,
        grid=(NQB,),
        in_specs=[
            pl.BlockSpec((NH, S, HD), lambda i: (0, 0, 0)),
            pl.BlockSpec((NC, S), lambda i: (0, 0)),
            pl.BlockSpec((NC, NC * NC), lambda i: (0, 0)),
            pl.BlockSpec((NC, NC * NC), lambda i: (0, 0)),
            pl.BlockSpec((NC * NC, NC), lambda i: (0, 0)),
            pl.BlockSpec((4 * QB, 3 * QB), lambda i: (0, 0)),
            pl.BlockSpec((QB, S), lambda i: (i, 0)),
            pl.BlockSpec((D, D), lambda i: (0, 0)),
        ],
        out_specs=pl.BlockSpec((QB, D), lambda i: (i, 0)),
        out_shape=jax.ShapeDtypeStruct((S, D), jnp.float32),
    )(proj, E, Tm, Rm, RmT, w3x4, win, WoT)

    return out.reshape(B, S, D)


# speedup vs baseline: 2.5686x; 1.4091x over previous
import jax
import jax.numpy as jnp
from jax.experimental import pallas as pl
from jax.experimental.pallas import tpu as pltpu

B, S, D = 1, 2048, 1024
HD = 64
STD_Q, STD_KV = 12, 3
HSA_H = 4
CHUNK, TOPK = 64, 8
SW, HSW = 512, 512
SCALE = HD ** -0.5
NC = S // CHUNK
QB = 256
NQB = S // QB
NH = STD_Q + STD_KV * 2 + HSA_H + 2
NEG = -1e9
HI = jax.lax.Precision.HIGHEST


def _rot_half(x):
    return jnp.concatenate([-x[:, HD // 2:], x[:, :HD // 2]], axis=1)


def _proj_kernel(x_ref, wt_ref, cos_ref, sin_ref, qn_ref, kn_ref, out_ref):
    x = x_ref[...]
    y = jnp.dot(x, wt_ref[...], preferred_element_type=jnp.float32)
    cos = cos_ref[...]
    sin = sin_ref[...]
    qn = qn_ref[...]
    kn = kn_ref[...]
    for h in range(NH):
        seg = y[:, h * HD:(h + 1) * HD]
        is_q = h < STD_Q or (18 <= h < 22)
        is_k = (STD_Q <= h < STD_Q + STD_KV) or h == 22
        if is_q or is_k:
            w = qn if is_q else kn
            seg = seg * jax.lax.rsqrt(
                jnp.mean(seg * seg, axis=1, keepdims=True) + 1e-6) * w
            seg = seg * cos + _rot_half(seg) * sin
        out_ref[h, :, :] = seg


def _fused_kernel(proj_ref, e_ref, w3_ref, win_ref, wot_ref, out_ref,
                  lm_ref):
    qb = pl.program_id(0)
    rows = pl.ds(qb * QB, QB)

    @pl.when(qb == 0)
    def _():
        hk = proj_ref[22]
        lm_ref[...] = jnp.dot(
            e_ref[...], hk, preferred_element_type=jnp.float32,
            precision=HI) * (1.0 / CHUNK)

    hq_cat = jnp.concatenate(
        [proj_ref[18 + h, rows, :] for h in range(HSA_H)], axis=0)
    x = jnp.dot(hq_cat, lm_ref[...].T, preferred_element_type=jnp.float32,
                precision=HI) * SCALE
    i_loc = qb * QB + jax.lax.broadcasted_iota(jnp.int32, (QB, 1), 0)
    off = (i_loc - (HSW - 1)) // CHUNK
    cid = jax.lax.broadcasted_iota(jnp.int32, (QB, NC), 1)
    avail = cid < off
    avail4 = jnp.concatenate([avail] * HSA_H, axis=0)
    xm = jnp.where(avail4, x, NEG)
    v = xm
    rem = jnp.full((HSA_H * QB, 1), float(TOPK), jnp.float32)
    thr = jnp.full((HSA_H * QB, 1), NEG, jnp.float32)
    for _ in range(TOPK):
        cur = jnp.max(v, axis=1, keepdims=True)
        eqm = v == cur
        n = jnp.sum(eqm.astype(jnp.float32), axis=1, keepdims=True)
        thr = jnp.where(rem > 0, cur, thr)
        v = jnp.where(eqm, -jnp.inf, v)
        rem = rem - n
    sel = ((xm >= thr) & avail4).astype(jnp.float32)

    outs = []
    w3 = w3_ref[...]
    v0 = (qb >= 2).astype(jnp.float32)
    v1 = (qb >= 1).astype(jnp.float32)
    cm = jnp.concatenate([jnp.full((1, QB), v0, jnp.float32),
                          jnp.full((1, QB), v1, jnp.float32),
                          jnp.ones((1, QB), jnp.float32)], axis=1)
    kbs = [pl.ds(jnp.maximum(qb - 2 + t, 0) * QB, QB) for t in range(3)]
    for g in range(STD_KV):
        qcat = jnp.concatenate(
            [proj_ref[4 * g + hh, rows, :] for hh in range(4)], axis=0)
        kband = jnp.concatenate(
            [proj_ref[STD_Q + g, kb, :] for kb in kbs], axis=0)
        vband = jnp.concatenate(
            [proj_ref[STD_Q + STD_KV + g, kb, :] for kb in kbs], axis=0)
        lg = jnp.dot(qcat, kband.T,
                     preferred_element_type=jnp.float32) * SCALE
        m = jnp.max(lg, axis=1, keepdims=True)
        p = jnp.exp(lg - m) * w3 * cm
        s = jnp.sum(p, axis=1, keepdims=True)
        o = jnp.dot(p, vband, preferred_element_type=jnp.float32) / s
        outs.extend(o[hh * QB:(hh + 1) * QB] for hh in range(4))

    kf = proj_ref[22]
    vf = proj_ref[23]
    winblk = win_ref[...]
    for h in range(HSA_H):
        qh = hq_cat[h * QB:(h + 1) * QB]
        lg = jnp.dot(qh, kf.T, preferred_element_type=jnp.float32) * SCALE
        tok = jnp.dot(sel[h * QB:(h + 1) * QB], e_ref[...],
                      preferred_element_type=jnp.float32)
        msk = jnp.maximum(winblk, tok)
        m = jnp.max(lg, axis=1, keepdims=True)
        p = jnp.exp(lg - m) * msk
        s = jnp.sum(p, axis=1, keepdims=True)
        outs.append(jnp.dot(p, vf, preferred_element_type=jnp.float32) / s)

    xcat = jnp.concatenate(outs, axis=1)
    out_ref[...] = jnp.dot(xcat, wot_ref[...],
                           preferred_element_type=jnp.float32)


def kernel(hidden_states, Wq, Wk, Wv, Whq, Whk, Whv, Wo, q_norm_w, k_norm_w):
    x = hidden_states.reshape(S, D)
    Wt = jnp.concatenate([Wq, Wk, Wv, Whq, Whk, Whv], axis=0).T

    pos = jnp.arange(S)
    inv = 1.0 / (10000.0 ** (jnp.arange(0, HD, 2).astype(jnp.float32) / HD))
    ang = pos[:, None] * inv[None, :]
    emb = jnp.concatenate([ang, ang], axis=-1)
    cos = jnp.cos(emb).astype(jnp.float32)
    sin = jnp.sin(emb).astype(jnp.float32)
    qn = q_norm_w.reshape(1, HD)
    kn = k_norm_w.reshape(1, HD)

    proj = pl.pallas_call(
        _proj_kernel,
        grid=(NQB,),
        in_specs=[
            pl.BlockSpec((QB, D), lambda i: (i, 0)),
            pl.BlockSpec((D, NH * HD), lambda i: (0, 0)),
            pl.BlockSpec((QB, HD), lambda i: (i, 0)),
            pl.BlockSpec((QB, HD), lambda i: (i, 0)),
            pl.BlockSpec((1, HD), lambda i: (0, 0)),
            pl.BlockSpec((1, HD), lambda i: (0, 0)),
        ],
        out_specs=pl.BlockSpec((NH, QB, HD), lambda i: (0, i, 0)),
        out_shape=jax.ShapeDtypeStruct((NH, S, HD), jnp.float32),
    )(x, Wt, cos, sin, qn, kn)

    jj = jnp.arange(S)
    cidx = jnp.arange(NC)
    E = (jj[None, :] // CHUNK == cidx[:, None]).astype(jnp.float32)
    r = jnp.arange(QB)
    col = jnp.arange(3 * QB)
    w3 = ((col[None, :] > r[:, None]) &
          (col[None, :] <= r[:, None] + SW)).astype(jnp.float32)
    w3x4 = jnp.tile(w3, (HSA_H, 1))
    ii = jnp.arange(S)
    win = ((jj[None, :] <= ii[:, None]) &
           (ii[:, None] - jj[None, :] < SW)).astype(jnp.float32)
    WoT = Wo.T

    out = pl.pallas_call(
        _fused_kernel,
        grid=(NQB,),
        in_specs=[
            pl.BlockSpec((NH, S, HD), lambda i: (0, 0, 0)),
            pl.BlockSpec((NC, S), lambda i: (0, 0)),
            pl.BlockSpec((4 * QB, 3 * QB), lambda i: (0, 0)),
            pl.BlockSpec((QB, S), lambda i: (i, 0)),
            pl.BlockSpec((D, D), lambda i: (0, 0)),
        ],
        out_specs=pl.BlockSpec((QB, D), lambda i: (i, 0)),
        out_shape=jax.ShapeDtypeStruct((S, D), jnp.float32),
        scratch_shapes=[pltpu.VMEM((NC, HD), jnp.float32)],
    )(proj, E, w3x4, win, WoT)

    return out.reshape(B, S, D)
